# baseline (speedup 1.0000x reference)
import functools

import jax
import jax.numpy as jnp
from jax import lax
from jax.experimental import pallas as pl
from jax.experimental.pallas import tpu as pltpu
from jax.experimental.pallas import tpu_sc as plsc

D_MODEL = 128
MAX_LEN = 100
N_EDGES = 320000

_NUM_CORES = 2
_NUM_SUBCORES = 16
_NW = _NUM_CORES * _NUM_SUBCORES
_B_PER_W = N_EDGES // _NW
_CH = 200
_NCH = _B_PER_W // _CH
_NBUF = 5
_LAG = 3

_mesh = plsc.VectorSubcoreMesh(core_axis_name="c", subcore_axis_name="s")


@functools.partial(
    pl.kernel,
    mesh=_mesh,
    out_type=jax.ShapeDtypeStruct((N_EDGES, D_MODEL), jnp.float32),
    scratch_types=(
        [pltpu.VMEM((_CH,), jnp.int32) for _ in range(_NBUF)]
        + [pltpu.VMEM((_CH, D_MODEL), jnp.float32) for _ in range(_NBUF)]
        + [pltpu.VMEM_SHARED((MAX_LEN, D_MODEL), jnp.float32)]
        + [pltpu.SemaphoreType.DMA for _ in range(3 * _NBUF)]
    ),
)
def _pe_gather(idx_hbm, table_hbm, out_hbm, *refs):
    idxs = refs[0:_NBUF]
    rowss = refs[_NBUF:2 * _NBUF]
    table_v = refs[2 * _NBUF]
    si = refs[2 * _NBUF + 1: 2 * _NBUF + 1 + _NBUF]
    sg = refs[2 * _NBUF + 1 + _NBUF: 2 * _NBUF + 1 + 2 * _NBUF]
    sw = refs[2 * _NBUF + 1 + 2 * _NBUF: 2 * _NBUF + 1 + 3 * _NBUF]

    wid = lax.axis_index("s") * _NUM_CORES + lax.axis_index("c")
    base = wid * _B_PER_W

    @pl.when(lax.axis_index("s") == 0)
    def _stage_table():
        pltpu.sync_copy(table_hbm, table_v)

    plsc.subcore_barrier()

    def fire_idx(slot, off):
        pltpu.async_copy(idx_hbm.at[pl.ds(off, _CH)], idxs[slot], si[slot])

    def wait_idx(slot, off):
        pltpu.make_async_copy(
            idx_hbm.at[pl.ds(off, _CH)], idxs[slot], si[slot]).wait()

    def fire_gather(slot):
        pltpu.async_copy(table_v.at[idxs[slot]], rowss[slot], sg[slot])

    def wait_gather(slot):
        pltpu.make_async_copy(
            table_v.at[idxs[slot]], rowss[slot], sg[slot]).wait()

    def fire_write(slot, off):
        pltpu.async_copy(rowss[slot], out_hbm.at[pl.ds(off, _CH)], sw[slot])

    def wait_write(slot, off):
        pltpu.make_async_copy(
            rowss[slot], out_hbm.at[pl.ds(off, _CH)], sw[slot]).wait()

    for b in range(_NBUF):
        fire_idx(b, base + b * _CH)

    for c in range(_LAG):
        wait_idx(c, base + c * _CH)
        fire_gather(c)
    for c in range(_LAG, _NBUF):
        b, bp = c % _NBUF, (c - _LAG) % _NBUF
        wait_idx(b, base + c * _CH)
        fire_gather(b)
        wait_gather(bp)
        fire_write(bp, base + (c - _LAG) * _CH)
        fire_idx(bp, base + (c + _NBUF - _LAG) * _CH)

    def round_(k, carry):
        for b in range(_NBUF):
            bp = (b - _LAG) % _NBUF
            c = k * _NBUF + b
            off = base + c * _CH
            wait_idx(b, off)
            wait_write(b, off - _NBUF * _CH)
            fire_gather(b)
            wait_gather(bp)
            fire_write(bp, off - _LAG * _CH)
            p = jnp.where(c + _NBUF - _LAG < _NCH, c + _NBUF - _LAG, c - _LAG)
            fire_idx(bp, base + p * _CH)
        return carry

    lax.fori_loop(1, _NCH // _NBUF, round_, 0)

    for c in range(_NCH - _LAG, _NCH):
        b = c % _NBUF
        wait_gather(b)
        fire_write(b, base + c * _CH)
    for c in range(_NCH - _NBUF, _NCH):
        b = c % _NBUF
        wait_write(b, base + c * _CH)
    for c in range(_NCH - _NBUF + _LAG, _NCH):
        bp = (c - _LAG) % _NBUF
        wait_idx(bp, base + (c - _LAG) * _CH)


def kernel(edge_type, pe):
    return _pe_gather(edge_type.astype(jnp.int32), pe)


# speedup vs baseline: 1.0090x; 1.0090x over previous
import functools

import jax
import jax.numpy as jnp
from jax import lax
from jax.experimental import pallas as pl
from jax.experimental.pallas import tpu as pltpu
from jax.experimental.pallas import tpu_sc as plsc

D_MODEL = 128
MAX_LEN = 100
N_EDGES = 320000

_NUM_CORES = 2
_NUM_SUBCORES = 16
_NW = _NUM_CORES * _NUM_SUBCORES
_B_PER_W = N_EDGES // _NW
_CH = 200
_NCH = _B_PER_W // _CH
_NBUF = 5
_LAG = 2

_mesh = plsc.VectorSubcoreMesh(core_axis_name="c", subcore_axis_name="s")


@functools.partial(
    pl.kernel,
    mesh=_mesh,
    out_type=jax.ShapeDtypeStruct((N_EDGES, D_MODEL), jnp.float32),
    scratch_types=(
        [pltpu.VMEM((_CH,), jnp.int32) for _ in range(_NBUF)]
        + [pltpu.VMEM((_CH, D_MODEL), jnp.float32) for _ in range(_NBUF)]
        + [pltpu.VMEM_SHARED((MAX_LEN, D_MODEL), jnp.float32)]
        + [pltpu.SemaphoreType.DMA for _ in range(3 * _NBUF)]
    ),
)
def _pe_gather(idx_hbm, table_hbm, out_hbm, *refs):
    idxs = refs[0:_NBUF]
    rowss = refs[_NBUF:2 * _NBUF]
    table_v = refs[2 * _NBUF]
    si = refs[2 * _NBUF + 1: 2 * _NBUF + 1 + _NBUF]
    sg = refs[2 * _NBUF + 1 + _NBUF: 2 * _NBUF + 1 + 2 * _NBUF]
    sw = refs[2 * _NBUF + 1 + 2 * _NBUF: 2 * _NBUF + 1 + 3 * _NBUF]

    wid = lax.axis_index("s") * _NUM_CORES + lax.axis_index("c")
    base = wid * _B_PER_W

    @pl.when(lax.axis_index("s") == 0)
    def _stage_table():
        pltpu.sync_copy(table_hbm, table_v)

    plsc.subcore_barrier()

    def fire_idx(slot, off):
        pltpu.async_copy(idx_hbm.at[pl.ds(off, _CH)], idxs[slot], si[slot])

    def wait_idx(slot, off):
        pltpu.make_async_copy(
            idx_hbm.at[pl.ds(off, _CH)], idxs[slot], si[slot]).wait()

    def fire_gather(slot):
        pltpu.async_copy(table_v.at[idxs[slot]], rowss[slot], sg[slot])

    def wait_gather(slot):
        pltpu.make_async_copy(
            table_v.at[idxs[slot]], rowss[slot], sg[slot]).wait()

    def fire_write(slot, off):
        pltpu.async_copy(rowss[slot], out_hbm.at[pl.ds(off, _CH)], sw[slot])

    def wait_write(slot, off):
        pltpu.make_async_copy(
            rowss[slot], out_hbm.at[pl.ds(off, _CH)], sw[slot]).wait()

    for b in range(_NBUF):
        fire_idx(b, base + b * _CH)

    for c in range(_LAG):
        wait_idx(c, base + c * _CH)
        fire_gather(c)
    for c in range(_LAG, _NBUF):
        b, bp = c % _NBUF, (c - _LAG) % _NBUF
        wait_idx(b, base + c * _CH)
        fire_gather(b)
        wait_gather(bp)
        fire_write(bp, base + (c - _LAG) * _CH)
        fire_idx(bp, base + (c + _NBUF - _LAG) * _CH)

    def round_(k, carry):
        for b in range(_NBUF):
            bp = (b - _LAG) % _NBUF
            c = k * _NBUF + b
            off = base + c * _CH
            wait_idx(b, off)
            wait_write(b, off - _NBUF * _CH)
            fire_gather(b)
            wait_gather(bp)
            fire_write(bp, off - _LAG * _CH)
            p = jnp.where(c + _NBUF - _LAG < _NCH, c + _NBUF - _LAG, c - _LAG)
            fire_idx(bp, base + p * _CH)
        return carry

    lax.fori_loop(1, _NCH // _NBUF, round_, 0)

    for c in range(_NCH - _LAG, _NCH):
        b = c % _NBUF
        wait_gather(b)
        fire_write(b, base + c * _CH)
    for c in range(_NCH - _NBUF, _NCH):
        b = c % _NBUF
        wait_write(b, base + c * _CH)
    for c in range(_NCH - _NBUF + _LAG, _NCH):
        bp = (c - _LAG) % _NBUF
        wait_idx(bp, base + (c - _LAG) * _CH)


def kernel(edge_type, pe):
    return _pe_gather(edge_type.astype(jnp.int32), pe)


# speedup vs baseline: 1.0189x; 1.0098x over previous
import functools

import jax
import jax.numpy as jnp
from jax import lax
from jax.experimental import pallas as pl
from jax.experimental.pallas import tpu as pltpu
from jax.experimental.pallas import tpu_sc as plsc

D_MODEL = 128
MAX_LEN = 100
N_EDGES = 320000

_NUM_CORES = 2
_NUM_SUBCORES = 16
_NW = _NUM_CORES * _NUM_SUBCORES
_B_PER_W = N_EDGES // _NW
_CH = 80
_NCH = _B_PER_W // _CH
_NBUF = 5
_LAG = 2

_mesh = plsc.VectorSubcoreMesh(core_axis_name="c", subcore_axis_name="s")


@functools.partial(
    pl.kernel,
    mesh=_mesh,
    out_type=jax.ShapeDtypeStruct((N_EDGES, D_MODEL), jnp.float32),
    scratch_types=(
        [pltpu.VMEM((_CH,), jnp.int32) for _ in range(_NBUF)]
        + [pltpu.VMEM((_CH, D_MODEL), jnp.float32) for _ in range(_NBUF)]
        + [pltpu.VMEM_SHARED((MAX_LEN, D_MODEL), jnp.float32)]
        + [pltpu.SemaphoreType.DMA for _ in range(3 * _NBUF)]
    ),
)
def _pe_gather(idx_hbm, table_hbm, out_hbm, *refs):
    idxs = refs[0:_NBUF]
    rowss = refs[_NBUF:2 * _NBUF]
    table_v = refs[2 * _NBUF]
    si = refs[2 * _NBUF + 1: 2 * _NBUF + 1 + _NBUF]
    sg = refs[2 * _NBUF + 1 + _NBUF: 2 * _NBUF + 1 + 2 * _NBUF]
    sw = refs[2 * _NBUF + 1 + 2 * _NBUF: 2 * _NBUF + 1 + 3 * _NBUF]

    wid = lax.axis_index("s") * _NUM_CORES + lax.axis_index("c")
    base = wid * _B_PER_W

    @pl.when(lax.axis_index("s") == 0)
    def _stage_table():
        pltpu.sync_copy(table_hbm, table_v)

    plsc.subcore_barrier()

    def fire_idx(slot, off):
        pltpu.async_copy(idx_hbm.at[pl.ds(off, _CH)], idxs[slot], si[slot])

    def wait_idx(slot, off):
        pltpu.make_async_copy(
            idx_hbm.at[pl.ds(off, _CH)], idxs[slot], si[slot]).wait()

    def fire_gather(slot):
        pltpu.async_copy(table_v.at[idxs[slot]], rowss[slot], sg[slot])

    def wait_gather(slot):
        pltpu.make_async_copy(
            table_v.at[idxs[slot]], rowss[slot], sg[slot]).wait()

    def fire_write(slot, off):
        pltpu.async_copy(rowss[slot], out_hbm.at[pl.ds(off, _CH)], sw[slot])

    def wait_write(slot, off):
        pltpu.make_async_copy(
            rowss[slot], out_hbm.at[pl.ds(off, _CH)], sw[slot]).wait()

    for b in range(_NBUF):
        fire_idx(b, base + b * _CH)

    for c in range(_LAG):
        wait_idx(c, base + c * _CH)
        fire_gather(c)
    for c in range(_LAG, _NBUF):
        b, bp = c % _NBUF, (c - _LAG) % _NBUF
        wait_idx(b, base + c * _CH)
        fire_gather(b)
        wait_gather(bp)
        fire_write(bp, base + (c - _LAG) * _CH)
        fire_idx(bp, base + (c + _NBUF - _LAG) * _CH)

    def round_(k, carry):
        for b in range(_NBUF):
            bp = (b - _LAG) % _NBUF
            c = k * _NBUF + b
            off = base + c * _CH
            wait_idx(b, off)
            wait_write(b, off - _NBUF * _CH)
            fire_gather(b)
            wait_gather(bp)
            fire_write(bp, off - _LAG * _CH)
            p = jnp.where(c + _NBUF - _LAG < _NCH, c + _NBUF - _LAG, c - _LAG)
            fire_idx(bp, base + p * _CH)
        return carry

    lax.fori_loop(1, _NCH // _NBUF, round_, 0)

    for c in range(_NCH - _LAG, _NCH):
        b = c % _NBUF
        wait_gather(b)
        fire_write(b, base + c * _CH)
    for c in range(_NCH - _NBUF, _NCH):
        b = c % _NBUF
        wait_write(b, base + c * _CH)
    for c in range(_NCH - _NBUF + _LAG, _NCH):
        bp = (c - _LAG) % _NBUF
        wait_idx(bp, base + (c - _LAG) * _CH)


def kernel(edge_type, pe):
    return _pe_gather(edge_type.astype(jnp.int32), pe)
